# baseline (speedup 1.0000x reference)
import functools

import jax
import jax.numpy as jnp
from jax import lax
from jax.experimental import pallas as pl
from jax.experimental.pallas import tpu as pltpu
from jax.experimental.pallas import tpu_sc as plsc

VOCAB = 1000000
BATCH = 16384
M = 100
VP = 1001472

NUM_WORKERS = 32
BPW = BATCH // NUM_WORKERS
IPW = BPW * M
GROUPS = BPW // 16

_mesh = plsc.VectorSubcoreMesh(core_axis_name="c", subcore_axis_name="s")


SEG = VP // 16
NCH = 4
CM = M // NCH
CIPW = CM * BPW


@functools.partial(
    pl.kernel,
    mesh=_mesh,
    out_type=jax.ShapeDtypeStruct((BATCH,), jnp.float32),
    scratch_types=[
        pltpu.VMEM((CIPW,), jnp.int32),
        pltpu.VMEM((CIPW,), jnp.int32),
        pltpu.VMEM((CIPW,), jnp.float32),
        pltpu.VMEM((CIPW,), jnp.float32),
        pltpu.VMEM((BPW,), jnp.float32),
        pltpu.VMEM_SHARED((VP,), jnp.float32),
        pltpu.SemaphoreType.DMA,
        pltpu.SemaphoreType.DMA,
        pltpu.SemaphoreType.DMA,
        pltpu.SemaphoreType.DMA,
    ],
)
def _emb_sum(
    idx_hbm, tbl_hbm, out_hbm,
    idx_v0, idx_v1, vals_v0, vals_v1, out_v, tbl_sh,
    si0, si1, sg0, sg1,
):
    wid = lax.axis_index("s") * 2 + lax.axis_index("c")
    sid = lax.axis_index("s")
    idx_bufs, vals_bufs = [idx_v0, idx_v1], [vals_v0, vals_v1]
    isems, gsems = [si0, si1], [sg0, sg1]

    def idx_dma(c):
        return pltpu.async_copy(
            idx_hbm.at[pl.ds(wid * IPW + c * CIPW, CIPW)],
            idx_bufs[c % 2],
            isems[c % 2],
        )

    idx_cps = [idx_dma(0)]
    pltpu.sync_copy(
        tbl_hbm.at[pl.ds(sid * SEG, SEG)], tbl_sh.at[pl.ds(sid * SEG, SEG)]
    )
    plsc.subcore_barrier()

    idx_cps[0].wait()
    gat_cps = [
        pltpu.async_copy(tbl_sh.at[idx_bufs[0]], vals_bufs[0], gsems[0])
    ]
    if NCH > 1:
        idx_cps.append(idx_dma(1))

    zero = jnp.zeros((16,), jnp.float32)
    accs = (zero,) * GROUPS
    for c in range(NCH):
        cb, nb = c % 2, (c + 1) % 2
        gat_cps[c].wait()
        if c + 1 < NCH:
            idx_cps[c + 1].wait()
            gat_cps.append(
                pltpu.async_copy(
                    tbl_sh.at[idx_bufs[nb]], vals_bufs[nb], gsems[nb]
                )
            )
            if c + 2 < NCH:
                idx_cps.append(idx_dma(c + 2))

        vals_v = vals_bufs[cb]

        def red_body(m, accs):
            base = m * BPW
            return tuple(
                accs[g] + vals_v[pl.ds(base + g * 16, 16)]
                for g in range(GROUPS)
            )

        accs = lax.fori_loop(0, CM, red_body, accs)

    for g in range(GROUPS):
        out_v[pl.ds(g * 16, 16)] = accs[g]

    pltpu.sync_copy(out_v, out_hbm.at[pl.ds(wid * BPW, BPW)])


def kernel(inputs, linear_weights):
    idx = (
        (inputs.astype(jnp.int32) - 1)
        .reshape(NUM_WORKERS, BPW, M)
        .transpose(0, 2, 1)
        .reshape(NUM_WORKERS * IPW)
    )
    tbl = jnp.pad(linear_weights.reshape(VOCAB), (0, VP - VOCAB))
    out = _emb_sum(idx, tbl)
    return out.reshape(BATCH, 1)


# speedup vs baseline: 1.8134x; 1.8134x over previous
import functools

import jax
import jax.numpy as jnp
from jax import lax
from jax.experimental import pallas as pl
from jax.experimental.pallas import tpu as pltpu
from jax.experimental.pallas import tpu_sc as plsc

VOCAB = 1000000
BATCH = 16384
M = 100
VP = 1001472

NUM_WORKERS = 32
BPW = BATCH // NUM_WORKERS
SEG = VP // 16
NCH = 4
RB = BPW // NCH
RGROUPS = RB // 16

_mesh = plsc.VectorSubcoreMesh(core_axis_name="c", subcore_axis_name="s")


@functools.partial(
    pl.kernel,
    mesh=_mesh,
    compiler_params=pltpu.CompilerParams(needs_layout_passes=False),
    out_type=jax.ShapeDtypeStruct((BATCH,), jnp.float32),
    scratch_types=[
        pltpu.VMEM((RB * M,), jnp.int32),
        pltpu.VMEM((RB * M,), jnp.int32),
        pltpu.VMEM((RB * M,), jnp.float32),
        pltpu.VMEM((RB * M,), jnp.float32),
        pltpu.VMEM((BPW,), jnp.float32),
        pltpu.VMEM_SHARED((VP,), jnp.float32),
        pltpu.SemaphoreType.DMA,
        pltpu.SemaphoreType.DMA,
        pltpu.SemaphoreType.DMA,
        pltpu.SemaphoreType.DMA,
    ],
)
def _emb_sum(
    idx_hbm, tbl_hbm, out_hbm,
    idx_v0, idx_v1, vals_v0, vals_v1, out_v, tbl_sh,
    si0, si1, sg0, sg1,
):
    wid = lax.axis_index("s") * 2 + lax.axis_index("c")
    sid = lax.axis_index("s")
    idx_bufs, vals_bufs = [idx_v0, idx_v1], [vals_v0, vals_v1]
    isems, gsems = [si0, si1], [sg0, sg1]

    def idx_dma(c):
        return pltpu.async_copy(
            idx_hbm.at[wid * NCH + c], idx_bufs[c % 2], isems[c % 2]
        )

    idx_cps = [idx_dma(0)]
    pltpu.sync_copy(
        tbl_hbm.at[pl.ds(sid * SEG, SEG)], tbl_sh.at[pl.ds(sid * SEG, SEG)]
    )
    plsc.subcore_barrier()

    idx_cps[0].wait()
    gat_cps = [
        pltpu.async_copy(tbl_sh.at[idx_bufs[0]], vals_bufs[0], gsems[0])
    ]
    idx_cps.append(idx_dma(1))

    row_iota = lax.iota(jnp.int32, 16) * M
    row_base = [g * 16 * M + row_iota for g in range(RGROUPS)]

    for c in range(NCH):
        cb, nb = c % 2, (c + 1) % 2
        gat_cps[c].wait()
        if c + 1 < NCH:
            idx_cps[c + 1].wait()
            gat_cps.append(
                pltpu.async_copy(
                    tbl_sh.at[idx_bufs[nb]], vals_bufs[nb], gsems[nb]
                )
            )
            if c + 2 < NCH:
                idx_cps.append(idx_dma(c + 2))

        vals_v = vals_bufs[cb]

        def red_body(m, accs):
            return tuple(
                accs[g] + plsc.load_gather(vals_v, [row_base[g] + m])
                for g in range(RGROUPS)
            )

        zero = jnp.zeros((16,), jnp.float32)
        accs = lax.fori_loop(0, M, red_body, (zero,) * RGROUPS)
        for g in range(RGROUPS):
            out_v[pl.ds(c * RB + g * 16, 16)] = accs[g]

    pltpu.sync_copy(out_v, out_hbm.at[pl.ds(wid * BPW, BPW)])


def kernel(inputs, linear_weights):
    idx = inputs.astype(jnp.int32).reshape(BATCH // RB, RB * M)
    tbl = jnp.pad(linear_weights, ((1, VP - VOCAB - 1), (0, 0))).reshape(VP)
    out = _emb_sum(idx, tbl)
    return out.reshape(BATCH, 1)
